# baseline (speedup 1.0000x reference)
import jax
import jax.numpy as jnp
from jax import lax
from jax.experimental import pallas as pl
from jax.experimental.pallas import tpu as pltpu
from jax.experimental.pallas import tpu_sc as plsc

_SMPL_PARENTS = [-1, 0, 0, 0, 1, 2, 3, 4, 5, 6, 7, 8, 9, 9, 9, 12, 13, 14,
                 16, 17, 18, 19, 20, 21]
_J = 24
_L = 128
_N = 64
_LOUT = 111
_NODES_G = _L * _J
_SRC_G = _NODES_G * 3
_OUT_G = _LOUT * _J * 3


def _stencils():
    children = [[] for _ in range(_J)]
    for u, p in enumerate(_SMPL_PARENTS):
        if p >= 0:
            children[p].append(u)
    sten = []
    for j in range(_J):
        s = [(0, j)]
        if _SMPL_PARENTS[j] >= 0:
            s.append((0, _SMPL_PARENTS[j]))
        for c in children[j]:
            s.append((0, c))
        for dt in (1, 2, 4, 8):
            s.append((-dt, j))
        sten.append(s)
    return sten


_STEN = _stencils()


def _sc_body(src_hbm, w_hbm, out_hbm, w_v, src_v, asT, adT,
             h0T, h1T, h2T, out_v):
    wid = lax.axis_index("s") * 2 + lax.axis_index("c")
    iota = jnp.arange(16, dtype=jnp.int32)
    iota72 = iota * 72

    pltpu.sync_copy(w_hbm, w_v)

    def wbc(k):
        return w_v[pl.ds(k * 16, 16)]

    vs = [wbc(i) for i in range(3)]
    vd = [wbc(3 + i) for i in range(3)]
    M = [[wbc(6 + 3 * r + c) for c in range(3)] for r in range(3)]
    cb = [wbc(15 + c) for c in range(3)]

    def graph_body(gi, carry):
        g = wid * 2 + gi
        pltpu.sync_copy(src_hbm.at[pl.ds(g * _SRC_G, _SRC_G)], src_v)
        def node_one(v):
            jn = v // 8
            f = v % 8
            t0 = f * 16
            i3 = iota72 + (t0 * 72 + jn * 3)
            pos = jn * _L + t0
            x0 = plsc.load_gather(src_v, [i3])
            x1 = plsc.load_gather(src_v, [i3 + 1])
            x2 = plsc.load_gather(src_v, [i3 + 2])
            asT[pl.ds(pos, 16)] = x0 * vs[0] + x1 * vs[1] + x2 * vs[2]
            adT[pl.ds(pos, 16)] = x0 * vd[0] + x1 * vd[1] + x2 * vd[2]
            h0T[pl.ds(pos, 16)] = x0 * M[0][0] + x1 * M[1][0] + x2 * M[2][0]
            h1T[pl.ds(pos, 16)] = x0 * M[0][1] + x1 * M[1][1] + x2 * M[2][1]
            h2T[pl.ds(pos, 16)] = x0 * M[0][2] + x1 * M[1][2] + x2 * M[2][2]

        def node_body(v, c2):
            node_one(v * 4)
            node_one(v * 4 + 1)
            node_one(v * 4 + 2)
            node_one(v * 4 + 3)
            return c2

        lax.fori_loop(0, 48, node_body, 0)

        def tree(vals):
            while len(vals) > 1:
                nxt = [vals[i] + vals[i + 1]
                       for i in range(0, len(vals) - 1, 2)]
                if len(vals) % 2:
                    nxt.append(vals[-1])
                vals = nxt
            return vals[0]

        def joint_block(j, t0, ob):
            sten = _STEN[j]
            ad = adT[pl.ds(j * _L + t0, 16)]
            alphas = []
            for (dt, jk) in sten:
                a = asT[pl.ds(jk * _L + dt + t0, 16)]
                t = a + ad
                alphas.append(jnp.maximum(t, 0.2 * t))
            es = [jnp.exp(a) for a in alphas]
            r = 1.0 / tree(es)
            for c, hT in enumerate((h0T, h1T, h2T)):
                terms = [e * hT[pl.ds(jk * _L + dt + t0, 16)]
                         for e, (dt, jk) in zip(es, sten)]
                acc = cb[c] + r * tree(terms)
                plsc.store_scatter(out_v, [iota72 + (ob + 3 * j + c)], acc)

        def blk_body(b, c2):
            t0 = jnp.minimum(16 + 32 * b, 95)
            ob = (t0 - 16) * 72
            for j in range(_J):
                joint_block(j, t0, ob)
                joint_block(j, t0 + 16, ob + 16 * 72)
            return c2

        lax.fori_loop(0, 4, blk_body, 0)
        pltpu.sync_copy(out_v, out_hbm.at[pl.ds(g * _OUT_G, _OUT_G)])
        return carry

    lax.fori_loop(0, 2, graph_body, 0)


@jax.jit
def kernel(src, W_gat, att_src, att_dst, bias_gat, W_post, b_post):
    v_s = W_gat @ att_src
    v_d = W_gat @ att_dst
    Mw = W_gat @ W_post
    cbias = bias_gat @ W_post + b_post
    w18 = jnp.concatenate([v_s, v_d, Mw.reshape(9), cbias]).astype(jnp.float32)
    wvec = jnp.repeat(w18, 16)

    mesh = plsc.VectorSubcoreMesh(core_axis_name="c", subcore_axis_name="s",
                                  num_cores=2, num_subcores=16)
    f = pl.kernel(
        _sc_body,
        out_type=jax.ShapeDtypeStruct((_N * _OUT_G,), jnp.float32),
        mesh=mesh,
        compiler_params=pltpu.CompilerParams(needs_layout_passes=False),
        scratch_types=[
            pltpu.VMEM((288,), jnp.float32),
            pltpu.VMEM((_SRC_G,), jnp.float32),
            pltpu.VMEM((_NODES_G,), jnp.float32),
            pltpu.VMEM((_NODES_G,), jnp.float32),
            pltpu.VMEM((_NODES_G,), jnp.float32),
            pltpu.VMEM((_NODES_G,), jnp.float32),
            pltpu.VMEM((_NODES_G,), jnp.float32),
            pltpu.VMEM((_OUT_G,), jnp.float32),
        ],
    )
    out_flat = f(src.reshape(-1).astype(jnp.float32), wvec)
    return out_flat.reshape(_N, _LOUT, _J * 3)


# speedup vs baseline: 1.1714x; 1.1714x over previous
import jax
import jax.numpy as jnp
from jax import lax
from jax.experimental import pallas as pl
from jax.experimental.pallas import tpu as pltpu
from jax.experimental.pallas import tpu_sc as plsc

_SMPL_PARENTS = [-1, 0, 0, 0, 1, 2, 3, 4, 5, 6, 7, 8, 9, 9, 9, 12, 13, 14,
                 16, 17, 18, 19, 20, 21]
_J = 24
_L = 128
_N = 64
_LOUT = 111
_NODES_G = _L * _J
_SRC_G = _NODES_G * 3
_OUT_G = _LOUT * _J * 3


def _stencils():
    children = [[] for _ in range(_J)]
    for u, p in enumerate(_SMPL_PARENTS):
        if p >= 0:
            children[p].append(u)
    sten = []
    for j in range(_J):
        s = [(0, j)]
        if _SMPL_PARENTS[j] >= 0:
            s.append((0, _SMPL_PARENTS[j]))
        for c in children[j]:
            s.append((0, c))
        for dt in (1, 2, 4, 8):
            s.append((-dt, j))
        sten.append(s)
    return sten


_STEN = _stencils()


def _sc_body(src_hbm, w_hbm, out_hbm, w_v, src_v, asT, adT,
             h0T, h1T, h2T, out_v):
    wid = lax.axis_index("s") * 2 + lax.axis_index("c")
    iota = jnp.arange(16, dtype=jnp.int32)
    iota72 = iota * 72

    pltpu.sync_copy(w_hbm, w_v)

    def wbc(k):
        return w_v[pl.ds(k * 16, 16)]

    vs = [wbc(i) for i in range(3)]
    vd = [wbc(3 + i) for i in range(3)]
    M = [[wbc(6 + 3 * r + c) for c in range(3)] for r in range(3)]
    cb = [wbc(15 + c) for c in range(3)]

    def graph_body(gi, carry):
        g = wid * 2 + gi
        pltpu.sync_copy(src_hbm.at[pl.ds(g * _SRC_G, _SRC_G)], src_v)
        def node_one(v):
            jn = v // 8
            f = v % 8
            t0 = f * 16
            i3 = iota72 + (t0 * 72 + jn * 3)
            pos = jn * _L + t0
            x0 = plsc.load_gather(src_v, [i3])
            x1 = plsc.load_gather(src_v, [i3 + 1])
            x2 = plsc.load_gather(src_v, [i3 + 2])
            asT[pl.ds(pos, 16)] = x0 * vs[0] + x1 * vs[1] + x2 * vs[2]
            adT[pl.ds(pos, 16)] = x0 * vd[0] + x1 * vd[1] + x2 * vd[2]
            h0T[pl.ds(pos, 16)] = x0 * M[0][0] + x1 * M[1][0] + x2 * M[2][0]
            h1T[pl.ds(pos, 16)] = x0 * M[0][1] + x1 * M[1][1] + x2 * M[2][1]
            h2T[pl.ds(pos, 16)] = x0 * M[0][2] + x1 * M[1][2] + x2 * M[2][2]

        def node_body(v, c2):
            node_one(v * 4)
            node_one(v * 4 + 1)
            node_one(v * 4 + 2)
            node_one(v * 4 + 3)
            return c2

        lax.fori_loop(0, 48, node_body, 0)

        def blk_body(b, c2):
            t0 = jnp.minimum(16 + 16 * b, _LOUT)
            ob = (t0 - 16) * 72
            for j in range(_J):
                sten = _STEN[j]
                ad = adT[pl.ds(j * _L + t0, 16)]
                alphas = []
                for (dt, jk) in sten:
                    a = asT[pl.ds(jk * _L + dt + t0, 16)]
                    t = a + ad
                    alphas.append(jnp.maximum(t, 0.2 * t))
                es = [jnp.exp(a) for a in alphas]

                def tree(vals):
                    while len(vals) > 1:
                        nxt = [vals[i] + vals[i + 1]
                               for i in range(0, len(vals) - 1, 2)]
                        if len(vals) % 2:
                            nxt.append(vals[-1])
                        vals = nxt
                    return vals[0]

                r = 1.0 / tree(es)
                for c, hT in enumerate((h0T, h1T, h2T)):
                    terms = [e * hT[pl.ds(jk * _L + dt + t0, 16)]
                             for e, (dt, jk) in zip(es, sten)]
                    acc = cb[c] + r * tree(terms)
                    plsc.store_scatter(out_v, [iota72 + (ob + 3 * j + c)], acc)
            return c2

        lax.fori_loop(0, 7, blk_body, 0)
        pltpu.sync_copy(out_v, out_hbm.at[pl.ds(g * _OUT_G, _OUT_G)])
        return carry

    lax.fori_loop(0, 2, graph_body, 0)


@jax.jit
def kernel(src, W_gat, att_src, att_dst, bias_gat, W_post, b_post):
    v_s = W_gat @ att_src
    v_d = W_gat @ att_dst
    Mw = W_gat @ W_post
    cbias = bias_gat @ W_post + b_post
    w18 = jnp.concatenate([v_s, v_d, Mw.reshape(9), cbias]).astype(jnp.float32)
    wvec = jnp.repeat(w18, 16)

    mesh = plsc.VectorSubcoreMesh(core_axis_name="c", subcore_axis_name="s",
                                  num_cores=2, num_subcores=16)
    f = pl.kernel(
        _sc_body,
        out_type=jax.ShapeDtypeStruct((_N * _OUT_G,), jnp.float32),
        mesh=mesh,
        compiler_params=pltpu.CompilerParams(needs_layout_passes=False),
        scratch_types=[
            pltpu.VMEM((288,), jnp.float32),
            pltpu.VMEM((_SRC_G,), jnp.float32),
            pltpu.VMEM((_NODES_G,), jnp.float32),
            pltpu.VMEM((_NODES_G,), jnp.float32),
            pltpu.VMEM((_NODES_G,), jnp.float32),
            pltpu.VMEM((_NODES_G,), jnp.float32),
            pltpu.VMEM((_NODES_G,), jnp.float32),
            pltpu.VMEM((_OUT_G,), jnp.float32),
        ],
    )
    out_flat = f(src.reshape(-1).astype(jnp.float32), wvec)
    return out_flat.reshape(_N, _LOUT, _J * 3)


# speedup vs baseline: 1.2099x; 1.0329x over previous
import jax
import jax.numpy as jnp
from jax import lax
from jax.experimental import pallas as pl
from jax.experimental.pallas import tpu as pltpu
from jax.experimental.pallas import tpu_sc as plsc

_SMPL_PARENTS = [-1, 0, 0, 0, 1, 2, 3, 4, 5, 6, 7, 8, 9, 9, 9, 12, 13, 14,
                 16, 17, 18, 19, 20, 21]
_J = 24
_L = 128
_N = 64
_LOUT = 111
_NODES_G = _L * _J
_SRC_G = _NODES_G * 3
_OUT_G = _LOUT * _J * 3


def _stencils():
    children = [[] for _ in range(_J)]
    for u, p in enumerate(_SMPL_PARENTS):
        if p >= 0:
            children[p].append(u)
    sten = []
    for j in range(_J):
        s = [(0, j)]
        if _SMPL_PARENTS[j] >= 0:
            s.append((0, _SMPL_PARENTS[j]))
        for c in children[j]:
            s.append((0, c))
        for dt in (1, 2, 4, 8):
            s.append((-dt, j))
        sten.append(s)
    return sten


_STEN = _stencils()

_ORDER = [0, 1, 4, 7, 10, 2, 5, 8, 11, 3, 6, 9, 12, 15,
          13, 16, 18, 20, 22, 14, 17, 19, 21, 23]


def _sc_body(src_hbm, w_hbm, out_hbm, w_v, src_v, asT, adT,
             h0T, h1T, h2T, out_v):
    wid = lax.axis_index("s") * 2 + lax.axis_index("c")
    iota = jnp.arange(16, dtype=jnp.int32)
    iota72 = iota * 72

    pltpu.sync_copy(w_hbm, w_v)

    def wbc(k):
        return w_v[pl.ds(k * 16, 16)]

    vs = [wbc(i) for i in range(3)]
    vd = [wbc(3 + i) for i in range(3)]
    M = [[wbc(6 + 3 * r + c) for c in range(3)] for r in range(3)]
    cb = [wbc(15 + c) for c in range(3)]

    def graph_body(gi, carry):
        g = wid * 2 + gi
        pltpu.sync_copy(src_hbm.at[pl.ds(g * _SRC_G, _SRC_G)], src_v)
        def node_one(v):
            jn = v // 8
            f = v % 8
            t0 = f * 16
            i3 = iota72 + (t0 * 72 + jn * 3)
            pos = jn * _L + t0
            x0 = plsc.load_gather(src_v, [i3])
            x1 = plsc.load_gather(src_v, [i3 + 1])
            x2 = plsc.load_gather(src_v, [i3 + 2])
            asT[pl.ds(pos, 16)] = x0 * vs[0] + x1 * vs[1] + x2 * vs[2]
            adT[pl.ds(pos, 16)] = x0 * vd[0] + x1 * vd[1] + x2 * vd[2]
            h0T[pl.ds(pos, 16)] = x0 * M[0][0] + x1 * M[1][0] + x2 * M[2][0]
            h1T[pl.ds(pos, 16)] = x0 * M[0][1] + x1 * M[1][1] + x2 * M[2][1]
            h2T[pl.ds(pos, 16)] = x0 * M[0][2] + x1 * M[1][2] + x2 * M[2][2]

        def node_body(v, c2):
            node_one(v * 4)
            node_one(v * 4 + 1)
            node_one(v * 4 + 2)
            node_one(v * 4 + 3)
            return c2

        lax.fori_loop(0, 48, node_body, 0)

        def tree(vals):
            while len(vals) > 1:
                nxt = [vals[i] + vals[i + 1]
                       for i in range(0, len(vals) - 1, 2)]
                if len(vals) % 2:
                    nxt.append(vals[-1])
                vals = nxt
            return vals[0]

        def blk_body(b, c2):
            t0 = jnp.minimum(16 + 16 * b, _LOUT)
            ob = (t0 - 16) * 72
            refs = (asT, h0T, h1T, h2T)
            cache = {}
            for pos, j in enumerate(_ORDER):
                sten = _STEN[j]

                def ld(ai, dt, jk, cur):
                    if dt == 0:
                        if (ai, jk) in cache:
                            cur[(ai, jk)] = cache[(ai, jk)]
                        elif (ai, jk) not in cur:
                            cur[(ai, jk)] = refs[ai][pl.ds(jk * _L + t0, 16)]
                        return cur[(ai, jk)]
                    return refs[ai][pl.ds(jk * _L + dt + t0, 16)]

                cur = {}
                ad = adT[pl.ds(j * _L + t0, 16)]
                alphas = []
                for (dt, jk) in sten:
                    a = ld(0, dt, jk, cur)
                    t = a + ad
                    alphas.append(jnp.maximum(t, 0.2 * t))
                es = [jnp.exp(a) for a in alphas]
                r = 1.0 / tree(es)
                for c, hT in enumerate((h0T, h1T, h2T)):
                    terms = [e * ld(1 + c, dt, jk, cur)
                             for e, (dt, jk) in zip(es, sten)]
                    acc = cb[c] + r * tree(terms)
                    plsc.store_scatter(out_v, [iota72 + (ob + 3 * j + c)], acc)
                if pos + 1 < _J:
                    nj = _ORDER[pos + 1]
                    keep = {nj} | {jk for (dt, jk) in _STEN[nj] if dt == 0}
                    cache = {k: v for k, v in cur.items() if k[1] in keep}
                else:
                    cache = {}
            return c2

        lax.fori_loop(0, 7, blk_body, 0)
        pltpu.sync_copy(out_v, out_hbm.at[pl.ds(g * _OUT_G, _OUT_G)])
        return carry

    lax.fori_loop(0, 2, graph_body, 0)


@jax.jit
def kernel(src, W_gat, att_src, att_dst, bias_gat, W_post, b_post):
    v_s = W_gat @ att_src
    v_d = W_gat @ att_dst
    Mw = W_gat @ W_post
    cbias = bias_gat @ W_post + b_post
    w18 = jnp.concatenate([v_s, v_d, Mw.reshape(9), cbias]).astype(jnp.float32)
    wvec = jnp.repeat(w18, 16)

    mesh = plsc.VectorSubcoreMesh(core_axis_name="c", subcore_axis_name="s",
                                  num_cores=2, num_subcores=16)
    f = pl.kernel(
        _sc_body,
        out_type=jax.ShapeDtypeStruct((_N * _OUT_G,), jnp.float32),
        mesh=mesh,
        compiler_params=pltpu.CompilerParams(needs_layout_passes=False),
        scratch_types=[
            pltpu.VMEM((288,), jnp.float32),
            pltpu.VMEM((_SRC_G,), jnp.float32),
            pltpu.VMEM((_NODES_G,), jnp.float32),
            pltpu.VMEM((_NODES_G,), jnp.float32),
            pltpu.VMEM((_NODES_G,), jnp.float32),
            pltpu.VMEM((_NODES_G,), jnp.float32),
            pltpu.VMEM((_NODES_G,), jnp.float32),
            pltpu.VMEM((_OUT_G,), jnp.float32),
        ],
    )
    out_flat = f(src.reshape(-1).astype(jnp.float32), wvec)
    return out_flat.reshape(_N, _LOUT, _J * 3)


# speedup vs baseline: 1.2288x; 1.0156x over previous
import jax
import jax.numpy as jnp
from jax import lax
from jax.experimental import pallas as pl
from jax.experimental.pallas import tpu as pltpu
from jax.experimental.pallas import tpu_sc as plsc

_SMPL_PARENTS = [-1, 0, 0, 0, 1, 2, 3, 4, 5, 6, 7, 8, 9, 9, 9, 12, 13, 14,
                 16, 17, 18, 19, 20, 21]
_J = 24
_L = 128
_N = 64
_LOUT = 111
_NODES_G = _L * _J
_SRC_G = _NODES_G * 3
_OUT_G = _LOUT * _J * 3


def _stencils():
    children = [[] for _ in range(_J)]
    for u, p in enumerate(_SMPL_PARENTS):
        if p >= 0:
            children[p].append(u)
    sten = []
    for j in range(_J):
        s = [(0, j)]
        if _SMPL_PARENTS[j] >= 0:
            s.append((0, _SMPL_PARENTS[j]))
        for c in children[j]:
            s.append((0, c))
        for dt in (1, 2, 4, 8):
            s.append((-dt, j))
        sten.append(s)
    return sten


_STEN = _stencils()

_ORDER = [0, 1, 4, 7, 10, 2, 5, 8, 11, 3, 6, 9, 12, 15,
          13, 16, 18, 20, 22, 14, 17, 19, 21, 23]


def _sc_body(src_hbm, w_hbm, out_hbm, w_v, src_v, asT, adT,
             h0T, h1T, h2T, out_v):
    wid = lax.axis_index("s") * 2 + lax.axis_index("c")
    iota = jnp.arange(16, dtype=jnp.int32)
    iota72 = iota * 72

    pltpu.sync_copy(w_hbm, w_v)

    def wbc(k):
        return w_v[pl.ds(k * 16, 16)]

    vs = [wbc(i) for i in range(3)]
    vd = [wbc(3 + i) for i in range(3)]
    M = [[wbc(6 + 3 * r + c) for c in range(3)] for r in range(3)]
    cb = [wbc(15 + c) for c in range(3)]

    def graph_body(gi, carry):
        g = wid * 2 + gi
        pltpu.sync_copy(src_hbm.at[pl.ds(g * _SRC_G, _SRC_G)], src_v)
        def node_one(v):
            jn = v // 8
            f = v % 8
            t0 = f * 16
            i3 = iota72 + (t0 * 72 + jn * 3)
            pos = jn * _L + t0
            x0 = plsc.load_gather(src_v, [i3])
            x1 = plsc.load_gather(src_v, [i3 + 1])
            x2 = plsc.load_gather(src_v, [i3 + 2])
            asT[pl.ds(pos, 16)] = x0 * vs[0] + x1 * vs[1] + x2 * vs[2]
            adT[pl.ds(pos, 16)] = x0 * vd[0] + x1 * vd[1] + x2 * vd[2]
            h0T[pl.ds(pos, 16)] = x0 * M[0][0] + x1 * M[1][0] + x2 * M[2][0]
            h1T[pl.ds(pos, 16)] = x0 * M[0][1] + x1 * M[1][1] + x2 * M[2][1]
            h2T[pl.ds(pos, 16)] = x0 * M[0][2] + x1 * M[1][2] + x2 * M[2][2]

        @plsc.parallel_loop(0, 48)
        def _node_loop(v):
            node_one(v * 4)
            node_one(v * 4 + 1)
            node_one(v * 4 + 2)
            node_one(v * 4 + 3)

        def tree(vals):
            while len(vals) > 1:
                nxt = [vals[i] + vals[i + 1]
                       for i in range(0, len(vals) - 1, 2)]
                if len(vals) % 2:
                    nxt.append(vals[-1])
                vals = nxt
            return vals[0]

        @plsc.parallel_loop(0, 7)
        def _blk_loop(b):
            t0 = jnp.minimum(16 + 16 * b, _LOUT)
            ob = (t0 - 16) * 72
            refs = (asT, h0T, h1T, h2T)
            cache = {}
            for pos, j in enumerate(_ORDER):
                sten = _STEN[j]

                def ld(ai, dt, jk, cur):
                    if dt == 0:
                        if (ai, jk) in cache:
                            cur[(ai, jk)] = cache[(ai, jk)]
                        elif (ai, jk) not in cur:
                            cur[(ai, jk)] = refs[ai][pl.ds(jk * _L + t0, 16)]
                        return cur[(ai, jk)]
                    return refs[ai][pl.ds(jk * _L + dt + t0, 16)]

                cur = {}
                ad = adT[pl.ds(j * _L + t0, 16)]
                alphas = []
                for (dt, jk) in sten:
                    a = ld(0, dt, jk, cur)
                    t = a + ad
                    alphas.append(jnp.maximum(t, 0.2 * t))
                es = [jnp.exp(a) for a in alphas]
                r = 1.0 / tree(es)
                for c, hT in enumerate((h0T, h1T, h2T)):
                    terms = [e * ld(1 + c, dt, jk, cur)
                             for e, (dt, jk) in zip(es, sten)]
                    acc = cb[c] + r * tree(terms)
                    plsc.store_scatter(out_v, [iota72 + (ob + 3 * j + c)], acc)
                if pos + 1 < _J:
                    nj = _ORDER[pos + 1]
                    keep = {nj} | {jk for (dt, jk) in _STEN[nj] if dt == 0}
                    cache = {k: v for k, v in cur.items() if k[1] in keep}
                else:
                    cache = {}
        pltpu.sync_copy(out_v, out_hbm.at[pl.ds(g * _OUT_G, _OUT_G)])
        return carry

    lax.fori_loop(0, 2, graph_body, 0)


@jax.jit
def kernel(src, W_gat, att_src, att_dst, bias_gat, W_post, b_post):
    v_s = W_gat @ att_src
    v_d = W_gat @ att_dst
    Mw = W_gat @ W_post
    cbias = bias_gat @ W_post + b_post
    w18 = jnp.concatenate([v_s, v_d, Mw.reshape(9), cbias]).astype(jnp.float32)
    wvec = jnp.repeat(w18, 16)

    mesh = plsc.VectorSubcoreMesh(core_axis_name="c", subcore_axis_name="s",
                                  num_cores=2, num_subcores=16)
    f = pl.kernel(
        _sc_body,
        out_type=jax.ShapeDtypeStruct((_N * _OUT_G,), jnp.float32),
        mesh=mesh,
        compiler_params=pltpu.CompilerParams(needs_layout_passes=False),
        scratch_types=[
            pltpu.VMEM((288,), jnp.float32),
            pltpu.VMEM((_SRC_G,), jnp.float32),
            pltpu.VMEM((_NODES_G,), jnp.float32),
            pltpu.VMEM((_NODES_G,), jnp.float32),
            pltpu.VMEM((_NODES_G,), jnp.float32),
            pltpu.VMEM((_NODES_G,), jnp.float32),
            pltpu.VMEM((_NODES_G,), jnp.float32),
            pltpu.VMEM((_OUT_G,), jnp.float32),
        ],
    )
    out_flat = f(src.reshape(-1).astype(jnp.float32), wvec)
    return out_flat.reshape(_N, _LOUT, _J * 3)
